# baseline (speedup 1.0000x reference)
import functools

import jax
import jax.numpy as jnp
from jax import lax
from jax.experimental import pallas as pl
from jax.experimental.pallas import tpu as pltpu
from jax.experimental.pallas import tpu_sc as plsc

Q, C, D = 4096, 32, 64
NC, NS, L = 2, 16, 16
NW = NC * NS
QPB = 4
NBT = Q // QPB
NB = NBT // NW
HB = QPB * C // 2
QB = 64
RB = QB * (C // 2)
INVERSE_SIGMA = 10.0

_mesh = plsc.VectorSubcoreMesh(core_axis_name="c", subcore_axis_name="s")


@functools.partial(
    pl.kernel,
    mesh=_mesh,
    compiler_params=pltpu.CompilerParams(
        needs_layout_passes=False, use_tc_tiling_on_sc=False),
    out_type=jax.ShapeDtypeStruct((Q * C // 2, 2 * D), jnp.float32),
    scratch_types=[
        pltpu.VMEM((NB, HB), jnp.int32),
        pltpu.VMEM((NB, HB), jnp.int32),
        pltpu.VMEM((2, HB, D), jnp.float32),
        pltpu.VMEM((2, HB, D), jnp.float32),
        pltpu.SemaphoreType.DMA((2,)),
    ],
)
def _sc_gather(ke_hbm, ko_hbm, X_hbm, out_hbm, ke_v, ko_v, re_v, ro_v, sem):
    wid = lax.axis_index("s") * NC + lax.axis_index("c")
    pltpu.sync_copy(ke_hbm.at[pl.ds(wid * NB, NB)], ke_v)
    pltpu.sync_copy(ko_hbm.at[pl.ds(wid * NB, NB)], ko_v)

    def gathers(b, slot):
        return (
            pltpu.make_async_copy(
                X_hbm.at[ke_v.at[b]], re_v.at[slot], sem.at[slot]),
            pltpu.make_async_copy(
                X_hbm.at[ko_v.at[b]], ro_v.at[slot], sem.at[slot]),
        )

    for cp in gathers(0, 0):
        cp.start()

    def body(i, carry):
        slot = lax.rem(i, 2)
        nxt = lax.rem(i + 1, 2)
        for cp in gathers(jnp.minimum(i + 1, NB - 1), nxt):
            cp.start()
        for cp in gathers(i, slot):
            cp.wait()
        r2 = (wid * NB + i) * HB
        pltpu.sync_copy(re_v.at[slot], out_hbm.at[pl.ds(r2, HB), pl.ds(0, D)])
        pltpu.sync_copy(ro_v.at[slot], out_hbm.at[pl.ds(r2, HB), pl.ds(D, D)])
        return carry

    lax.fori_loop(0, NB, body, 0, unroll=False)
    for cp in gathers(NB - 1, lax.rem(NB, 2)):
        cp.wait()


def _expm1s(z):
    return jnp.where(jnp.abs(z) < 1e-3, z + 0.5 * z * z, jnp.exp(z) - 1.0)


def _tc_body(xk_ref, x_ref, v_ref, o_ref):
    f32 = jnp.float32
    xk = xk_ref[...]
    x2 = x_ref[...]
    v2 = v_ref[...]
    H = C // 2

    def rows(a):
        w = a.shape[-1]
        return jnp.reshape(jnp.broadcast_to(a[:, None, :], (QB, H, w)), (RB, w))

    delta = xk - rows(x2)
    prod = delta * rows(v2)
    d2 = delta * delta
    lane = lax.broadcasted_iota(jnp.int32, (RB, 128), 1)
    left = lane < D
    hind = (lax.broadcasted_iota(jnp.int32, (128, 2), 1)
            == lax.broadcasted_iota(jnp.int32, (128, 2), 0) // D).astype(f32)
    mm = lambda a, b: jax.lax.dot(a, b, precision=jax.lax.Precision.HIGHEST,
                                  preferred_element_type=f32)
    dd = mm(prod, hind)
    nn = mm(d2, hind)
    dl, dr = dd[:, 0:1], dd[:, 1:2]
    nl, nr = nn[:, 0:1], nn[:, 1:2]
    laneq = lax.broadcasted_iota(jnp.int32, (QB, 128), 1)
    nv2 = jnp.sum(jnp.where(laneq < D, v2 * v2, 0.0), axis=1, keepdims=True)
    nv2e = rows(nv2)
    sl = dl / jnp.maximum(jnp.sqrt(nl * nv2e), 1e-8)
    sr = dr / jnp.maximum(jnp.sqrt(nr * nv2e), 1e-8)
    tl = _expm1s(INVERSE_SIGMA * sl)
    tr = _expm1s(INVERSE_SIGMA * sr)

    def seg_sum(a):
        return jnp.sum(jnp.reshape(a, (QB, H, a.shape[-1])), axis=1)

    mean = seg_sum(tl + tr) * (1.0 / C)
    invs = 1.0 / seg_sum(jnp.abs(tl) + jnp.abs(tr))
    wl = (tl - rows(mean)) * rows(invs)
    wr = (tr - rows(mean)) * rows(invs)
    wfull = jnp.where(left, wl, wr)
    res = seg_sum(xk * wfull)
    o_ref[...] = res[:, :D] + res[:, D:]


def _tc_compute(xk2, x2, v2):
    return pl.pallas_call(
        _tc_body,
        grid=(Q // QB,),
        in_specs=[
            pl.BlockSpec((RB, 2 * D), lambda b: (b, 0)),
            pl.BlockSpec((QB, 2 * D), lambda b: (b, 0)),
            pl.BlockSpec((QB, 2 * D), lambda b: (b, 0)),
        ],
        out_specs=pl.BlockSpec((QB, D), lambda b: (b, 0)),
        out_shape=jax.ShapeDtypeStruct((Q, D), jnp.float32),
    )(xk2, x2, v2)


def kernel(x, v, k, X):
    k2 = k.astype(jnp.int32).reshape(NBT, QPB * C)
    ke = k2[:, 0::2]
    ko = k2[:, 1::2]
    xk2 = _sc_gather(ke, ko, X)
    x2 = jnp.concatenate([x, x], axis=1)
    v2 = jnp.concatenate([v, v], axis=1)
    return _tc_compute(xk2, x2, v2)


# speedup vs baseline: 1.1313x; 1.1313x over previous
import functools

import jax
import jax.numpy as jnp
from jax import lax
from jax.experimental import pallas as pl
from jax.experimental.pallas import tpu as pltpu
from jax.experimental.pallas import tpu_sc as plsc

Q, C, D = 4096, 32, 64
NC, NS, L = 2, 16, 16
NW = NC * NS
QPB = 4
NBT = Q // QPB
NB = NBT // NW
HB = QPB * C // 2
QB = 64
RB = QB * (C // 2)
INVERSE_SIGMA = 10.0

_mesh = plsc.VectorSubcoreMesh(core_axis_name="c", subcore_axis_name="s")


@functools.partial(
    pl.kernel,
    mesh=_mesh,
    compiler_params=pltpu.CompilerParams(
        needs_layout_passes=False, use_tc_tiling_on_sc=False),
    out_type=jax.ShapeDtypeStruct((Q * C // 2, 2 * D), jnp.float32),
    scratch_types=[
        pltpu.VMEM((NB, HB), jnp.int32),
        pltpu.VMEM((NB, HB), jnp.int32),
        pltpu.VMEM((2, HB, D), jnp.float32),
        pltpu.VMEM((2, HB, D), jnp.float32),
        pltpu.SemaphoreType.DMA((2,)),
    ],
)
def _sc_gather(ke_hbm, ko_hbm, X_hbm, out_hbm, ke_v, ko_v, re_v, ro_v, sem):
    wid = lax.axis_index("s") * NC + lax.axis_index("c")
    pltpu.sync_copy(ke_hbm.at[pl.ds(wid * NB, NB)], ke_v)
    pltpu.sync_copy(ko_hbm.at[pl.ds(wid * NB, NB)], ko_v)

    def gathers(b, slot):
        return (
            pltpu.make_async_copy(
                X_hbm.at[ke_v.at[b]], re_v.at[slot], sem.at[slot]),
            pltpu.make_async_copy(
                X_hbm.at[ko_v.at[b]], ro_v.at[slot], sem.at[slot]),
        )

    for cp in gathers(0, 0):
        cp.start()

    def body(i, carry):
        slot = lax.rem(i, 2)
        nxt = lax.rem(i + 1, 2)
        for cp in gathers(jnp.minimum(i + 1, NB - 1), nxt):
            cp.start()
        for cp in gathers(i, slot):
            cp.wait()
        r2 = (wid * NB + i) * HB
        pltpu.sync_copy(re_v.at[slot], out_hbm.at[pl.ds(r2, HB), pl.ds(0, D)])
        pltpu.sync_copy(ro_v.at[slot], out_hbm.at[pl.ds(r2, HB), pl.ds(D, D)])
        return carry

    lax.fori_loop(0, NB, body, 0, unroll=False)
    for cp in gathers(NB - 1, lax.rem(NB, 2)):
        cp.wait()


def _expm1s(z):
    return jnp.where(jnp.abs(z) < 1e-3, z + 0.5 * z * z, jnp.exp(z) - 1.0)


def _tc_body(xk_ref, x_ref, v_ref, o_ref):
    f32 = jnp.float32
    xk = xk_ref[...]
    x2 = x_ref[...]
    v2 = v_ref[...]
    H = C // 2

    def rows(a):
        w = a.shape[-1]
        return jnp.reshape(jnp.broadcast_to(a[:, None, :], (QB, H, w)), (RB, w))

    delta = xk - rows(x2)
    prod = delta * rows(v2)
    d2 = delta * delta
    lane = lax.broadcasted_iota(jnp.int32, (RB, 128), 1)
    left = lane < D
    dl = jnp.sum(jnp.where(left, prod, 0.0), axis=1, keepdims=True)
    dr = jnp.sum(jnp.where(left, 0.0, prod), axis=1, keepdims=True)
    nl = jnp.sum(jnp.where(left, d2, 0.0), axis=1, keepdims=True)
    nr = jnp.sum(jnp.where(left, 0.0, d2), axis=1, keepdims=True)
    laneq = lax.broadcasted_iota(jnp.int32, (QB, 128), 1)
    nv2 = jnp.sum(jnp.where(laneq < D, v2 * v2, 0.0), axis=1, keepdims=True)
    dd = jnp.concatenate([dl, dr], axis=1)
    nn = jnp.concatenate([nl, nr], axis=1)
    s = dd / jnp.maximum(jnp.sqrt(nn * rows(nv2)), 1e-8)
    t = _expm1s(INVERSE_SIGMA * s)

    def seg_sum(a):
        return jnp.sum(jnp.reshape(a, (QB, H, a.shape[-1])), axis=1)

    tsum = jnp.sum(t, axis=1, keepdims=True)
    mean = seg_sum(tsum) * (1.0 / C)
    invs = 1.0 / seg_sum(jnp.sum(jnp.abs(t), axis=1, keepdims=True))
    w = (t - rows(mean)) * rows(invs)
    wfull = jnp.where(left, w[:, 0:1], w[:, 1:2])
    res = seg_sum(xk * wfull)
    o_ref[...] = res[:, :D] + res[:, D:]


def _tc_compute(xk2, x2, v2):
    return pl.pallas_call(
        _tc_body,
        grid=(Q // QB,),
        in_specs=[
            pl.BlockSpec((RB, 2 * D), lambda b: (b, 0)),
            pl.BlockSpec((QB, 2 * D), lambda b: (b, 0)),
            pl.BlockSpec((QB, 2 * D), lambda b: (b, 0)),
        ],
        out_specs=pl.BlockSpec((QB, D), lambda b: (b, 0)),
        out_shape=jax.ShapeDtypeStruct((Q, D), jnp.float32),
    )(xk2, x2, v2)


def kernel(x, v, k, X):
    k2 = k.astype(jnp.int32).reshape(NBT, QPB * C)
    ke = k2[:, 0::2]
    ko = k2[:, 1::2]
    xk2 = _sc_gather(ke, ko, X)
    x2 = jnp.concatenate([x, x], axis=1)
    v2 = jnp.concatenate([v, v], axis=1)
    return _tc_compute(xk2, x2, v2)


# speedup vs baseline: 1.1686x; 1.0329x over previous
import functools

import jax
import jax.numpy as jnp
from jax import lax
from jax.experimental import pallas as pl
from jax.experimental.pallas import tpu as pltpu
from jax.experimental.pallas import tpu_sc as plsc

Q, C, D = 4096, 32, 64
NC, NS, L = 2, 16, 16
NW = NC * NS
QPB = 4
NBT = Q // QPB
NB = NBT // NW
HB = QPB * C // 2
QB = 128
RB = QB * (C // 2)
INVERSE_SIGMA = 10.0

_mesh = plsc.VectorSubcoreMesh(core_axis_name="c", subcore_axis_name="s")


@functools.partial(
    pl.kernel,
    mesh=_mesh,
    compiler_params=pltpu.CompilerParams(
        needs_layout_passes=False, use_tc_tiling_on_sc=False),
    out_type=jax.ShapeDtypeStruct((Q * C // 2, 2 * D), jnp.float32),
    scratch_types=[
        pltpu.VMEM((NB, HB), jnp.int32),
        pltpu.VMEM((NB, HB), jnp.int32),
        pltpu.VMEM((2, HB, D), jnp.float32),
        pltpu.VMEM((2, HB, D), jnp.float32),
        pltpu.SemaphoreType.DMA((2,)),
    ],
)
def _sc_gather(ke_hbm, ko_hbm, X_hbm, out_hbm, ke_v, ko_v, re_v, ro_v, sem):
    wid = lax.axis_index("s") * NC + lax.axis_index("c")
    pltpu.sync_copy(ke_hbm.at[pl.ds(wid * NB, NB)], ke_v)
    pltpu.sync_copy(ko_hbm.at[pl.ds(wid * NB, NB)], ko_v)

    def gathers(b, slot):
        return (
            pltpu.make_async_copy(
                X_hbm.at[ke_v.at[b]], re_v.at[slot], sem.at[slot]),
            pltpu.make_async_copy(
                X_hbm.at[ko_v.at[b]], ro_v.at[slot], sem.at[slot]),
        )

    for cp in gathers(0, 0):
        cp.start()

    def body(i, carry):
        slot = lax.rem(i, 2)
        nxt = lax.rem(i + 1, 2)
        for cp in gathers(jnp.minimum(i + 1, NB - 1), nxt):
            cp.start()
        for cp in gathers(i, slot):
            cp.wait()
        r2 = (wid * NB + i) * HB
        pltpu.sync_copy(re_v.at[slot], out_hbm.at[pl.ds(r2, HB), pl.ds(0, D)])
        pltpu.sync_copy(ro_v.at[slot], out_hbm.at[pl.ds(r2, HB), pl.ds(D, D)])
        return carry

    lax.fori_loop(0, NB, body, 0, unroll=False)
    for cp in gathers(NB - 1, lax.rem(NB, 2)):
        cp.wait()


def _expm1s(z):
    return jnp.where(jnp.abs(z) < 1e-3, z + 0.5 * z * z, jnp.exp(z) - 1.0)


def _tc_body(xk_ref, x_ref, v_ref, o_ref):
    f32 = jnp.float32
    xk = xk_ref[...]
    x2 = x_ref[...]
    v2 = v_ref[...]
    H = C // 2

    def rows(a):
        w = a.shape[-1]
        return jnp.reshape(jnp.broadcast_to(a[:, None, :], (QB, H, w)), (RB, w))

    delta = xk - rows(x2)
    prod = delta * rows(v2)
    d2 = delta * delta
    lane = lax.broadcasted_iota(jnp.int32, (RB, 128), 1)
    left = lane < D
    dl = jnp.sum(jnp.where(left, prod, 0.0), axis=1, keepdims=True)
    dr = jnp.sum(jnp.where(left, 0.0, prod), axis=1, keepdims=True)
    nl = jnp.sum(jnp.where(left, d2, 0.0), axis=1, keepdims=True)
    nr = jnp.sum(jnp.where(left, 0.0, d2), axis=1, keepdims=True)
    laneq = lax.broadcasted_iota(jnp.int32, (QB, 128), 1)
    nv2 = jnp.sum(jnp.where(laneq < D, v2 * v2, 0.0), axis=1, keepdims=True)
    dd = jnp.concatenate([dl, dr], axis=1)
    nn = jnp.concatenate([nl, nr], axis=1)
    s = dd / jnp.maximum(jnp.sqrt(nn * rows(nv2)), 1e-8)
    t = _expm1s(INVERSE_SIGMA * s)

    def seg_sum(a):
        return jnp.sum(jnp.reshape(a, (QB, H, a.shape[-1])), axis=1)

    tsum = jnp.sum(t, axis=1, keepdims=True)
    mean = seg_sum(tsum) * (1.0 / C)
    invs = 1.0 / seg_sum(jnp.sum(jnp.abs(t), axis=1, keepdims=True))
    w = (t - rows(mean)) * rows(invs)
    wfull = jnp.where(left, w[:, 0:1], w[:, 1:2])
    res = seg_sum(xk * wfull)
    o_ref[...] = res[:, :D] + res[:, D:]


def _tc_compute(xk2, x2, v2):
    return pl.pallas_call(
        _tc_body,
        grid=(Q // QB,),
        in_specs=[
            pl.BlockSpec((RB, 2 * D), lambda b: (b, 0)),
            pl.BlockSpec((QB, 2 * D), lambda b: (b, 0)),
            pl.BlockSpec((QB, 2 * D), lambda b: (b, 0)),
        ],
        out_specs=pl.BlockSpec((QB, D), lambda b: (b, 0)),
        out_shape=jax.ShapeDtypeStruct((Q, D), jnp.float32),
    )(xk2, x2, v2)


def kernel(x, v, k, X):
    k2 = k.astype(jnp.int32).reshape(NBT, QPB * C)
    ke = k2[:, 0::2]
    ko = k2[:, 1::2]
    xk2 = _sc_gather(ke, ko, X)
    x2 = jnp.concatenate([x, x], axis=1)
    v2 = jnp.concatenate([v, v], axis=1)
    return _tc_compute(xk2, x2, v2)
